# baseline (device time: 135815 ns/iter reference)
import jax
import jax.numpy as jnp
from jax import lax
from jax.experimental import pallas as pl
from jax.experimental.pallas import tpu as pltpu


def kernel(ids, E):
    v_loc, d = E.shape
    t = ids.shape[0]
    t_half = t // 2

    my_x = lax.axis_index("x")
    my_y = lax.axis_index("y")

    ids_half = lax.dynamic_slice(ids, (my_y * t_half,), (t_half,))
    local = ids_half - my_x * v_loc
    mask = (local >= 0) & (local < v_loc)
    rows = jnp.take(E, jnp.clip(local, 0, v_loc - 1), axis=0)
    partial = jnp.where(mask[:, None], rows, jnp.float32(0))

    def body(part_ref, out_ref, xrecv_ref, ysend_ref, yrecv_ref,
             send_sems, recv_sems):
        mx = lax.axis_index("x")
        my = lax.axis_index("y")
        x_nbr = (1 - mx, my)
        y_nbr = (mx, 1 - my)

        barrier_sem = pltpu.get_barrier_semaphore()
        for nbr in (x_nbr, y_nbr):
            pl.semaphore_signal(
                barrier_sem, inc=1,
                device_id=nbr, device_id_type=pl.DeviceIdType.MESH,
            )
        pl.semaphore_wait(barrier_sem, 2)

        rdma_x = pltpu.make_async_remote_copy(
            src_ref=part_ref,
            dst_ref=xrecv_ref,
            send_sem=send_sems.at[0],
            recv_sem=recv_sems.at[0],
            device_id=x_nbr,
            device_id_type=pl.DeviceIdType.MESH,
        )
        rdma_x.start()
        rdma_x.wait()

        reduced = part_ref[...] + xrecv_ref[...]
        out_ref[pl.ds(my * t_half, t_half), :] = reduced
        ysend_ref[...] = reduced

        rdma_y = pltpu.make_async_remote_copy(
            src_ref=ysend_ref,
            dst_ref=yrecv_ref,
            send_sem=send_sems.at[1],
            recv_sem=recv_sems.at[1],
            device_id=y_nbr,
            device_id_type=pl.DeviceIdType.MESH,
        )
        rdma_y.start()
        rdma_y.wait()

        out_ref[pl.ds((1 - my) * t_half, t_half), :] = yrecv_ref[...]

    return pl.pallas_call(
        body,
        out_shape=jax.ShapeDtypeStruct((t, d), jnp.float32),
        in_specs=[pl.BlockSpec(memory_space=pltpu.VMEM)],
        out_specs=pl.BlockSpec(memory_space=pltpu.VMEM),
        scratch_shapes=[
            pltpu.VMEM((t_half, d), jnp.float32),
            pltpu.VMEM((t_half, d), jnp.float32),
            pltpu.VMEM((t_half, d), jnp.float32),
            pltpu.SemaphoreType.DMA((2,)),
            pltpu.SemaphoreType.DMA((2,)),
        ],
        compiler_params=pltpu.CompilerParams(collective_id=0),
    )(partial)


# device time: 96000 ns/iter; 1.4147x vs baseline; 1.4147x over previous
import jax
import jax.numpy as jnp
from jax import lax
from jax.experimental import pallas as pl
from jax.experimental.pallas import tpu as pltpu

N_CHUNKS = 8


def kernel(ids, E):
    v_loc, d = E.shape
    t = ids.shape[0]
    t_half = t // 2
    r = t_half // N_CHUNKS

    my_x = lax.axis_index("x")
    my_y = lax.axis_index("y")

    ids_half = lax.dynamic_slice(ids, (my_y * t_half,), (t_half,))
    local = ids_half - my_x * v_loc
    mask = (local >= 0) & (local < v_loc)
    rows = jnp.take(E, jnp.clip(local, 0, v_loc - 1), axis=0)
    partial = jnp.where(mask[:, None], rows, jnp.float32(0))

    def body(part_ref, out_ref, xrecv_ref,
             xsend_sems, xrecv_sems, ysend_sems, yrecv_sems):
        mx = lax.axis_index("x")
        my = lax.axis_index("y")
        x_nbr = (1 - mx, my)
        y_nbr = (mx, 1 - my)

        barrier_sem = pltpu.get_barrier_semaphore()
        for nbr in (x_nbr, y_nbr):
            pl.semaphore_signal(
                barrier_sem, inc=1,
                device_id=nbr, device_id_type=pl.DeviceIdType.MESH,
            )
        pl.semaphore_wait(barrier_sem, 2)

        my_half = my * t_half
        other_half = (1 - my) * t_half

        x_rdmas = []
        for k in range(N_CHUNKS):
            sl = pl.ds(k * r, r)
            rdma = pltpu.make_async_remote_copy(
                src_ref=part_ref.at[sl],
                dst_ref=xrecv_ref.at[sl],
                send_sem=xsend_sems.at[k],
                recv_sem=xrecv_sems.at[k],
                device_id=x_nbr,
                device_id_type=pl.DeviceIdType.MESH,
            )
            rdma.start()
            x_rdmas.append(rdma)

        y_sends = []
        y_recvs = []
        for k in range(N_CHUNKS):
            x_rdmas[k].wait_recv()
            sl = pl.ds(k * r, r)
            out_sl = pl.ds(my_half + k * r, r)
            out_ref[out_sl, :] = part_ref[sl, :] + xrecv_ref[sl, :]
            send = pltpu.make_async_remote_copy(
                src_ref=out_ref.at[out_sl],
                dst_ref=out_ref.at[out_sl],
                send_sem=ysend_sems.at[k],
                recv_sem=yrecv_sems.at[k],
                device_id=y_nbr,
                device_id_type=pl.DeviceIdType.MESH,
            )
            send.start()
            y_sends.append(send)
            recv = pltpu.make_async_remote_copy(
                src_ref=out_ref.at[out_sl],
                dst_ref=out_ref.at[pl.ds(other_half + k * r, r)],
                send_sem=ysend_sems.at[k],
                recv_sem=yrecv_sems.at[k],
                device_id=y_nbr,
                device_id_type=pl.DeviceIdType.MESH,
            )
            y_recvs.append(recv)

        for k in range(N_CHUNKS):
            y_recvs[k].wait_recv()
        for k in range(N_CHUNKS):
            x_rdmas[k].wait_send()
            y_sends[k].wait_send()

    return pl.pallas_call(
        body,
        out_shape=jax.ShapeDtypeStruct((t, d), jnp.float32),
        in_specs=[pl.BlockSpec(memory_space=pltpu.VMEM)],
        out_specs=pl.BlockSpec(memory_space=pltpu.VMEM),
        scratch_shapes=[
            pltpu.VMEM((t_half, d), jnp.float32),
            pltpu.SemaphoreType.DMA((N_CHUNKS,)),
            pltpu.SemaphoreType.DMA((N_CHUNKS,)),
            pltpu.SemaphoreType.DMA((N_CHUNKS,)),
            pltpu.SemaphoreType.DMA((N_CHUNKS,)),
        ],
        compiler_params=pltpu.CompilerParams(collective_id=0),
    )(partial)


# device time: 83741 ns/iter; 1.6218x vs baseline; 1.1464x over previous
import jax
import jax.numpy as jnp
from jax import lax
from jax.experimental import pallas as pl
from jax.experimental.pallas import tpu as pltpu

N_CHUNKS = 8


def kernel(ids, E):
    v_loc, d = E.shape
    t = ids.shape[0]
    t_half = t // 2
    r = t_half // N_CHUNKS

    my_x = lax.axis_index("x")
    my_y = lax.axis_index("y")

    ids_half = lax.dynamic_slice(ids, (my_y * t_half,), (t_half,))
    local = ids_half - my_x * v_loc
    clamped = jnp.clip(local, 0, v_loc - 1).astype(jnp.int32)
    maskf = ((local >= 0) & (local < v_loc)).astype(jnp.float32)[:, None]

    def body(idx_ref, maskf_ref, e_ref, out_ref, part_ref, xrecv_ref,
             gather_sems, xsend_sems, xrecv_sems, ysend_sems, yrecv_sems):
        mx = lax.axis_index("x")
        my = lax.axis_index("y")
        x_nbr = (1 - mx, my)
        y_nbr = (mx, 1 - my)

        def row_copy(row_idx, dst_row, sem):
            return pltpu.make_async_copy(
                e_ref.at[pl.ds(row_idx, 1)],
                part_ref.at[pl.ds(dst_row, 1)],
                sem,
            )

        def issue_gather(k):
            def fi(i, carry):
                row_copy(idx_ref[k * r + i], k * r + i,
                         gather_sems.at[k]).start()
                return carry
            lax.fori_loop(0, r, fi, 0)

        def wait_gather(k):
            def fi(i, carry):
                row_copy(0, 0, gather_sems.at[k]).wait()
                return carry
            lax.fori_loop(0, r, fi, 0)

        barrier_sem = pltpu.get_barrier_semaphore()
        for nbr in (x_nbr, y_nbr):
            pl.semaphore_signal(
                barrier_sem, inc=1,
                device_id=nbr, device_id_type=pl.DeviceIdType.MESH,
            )
        pl.semaphore_wait(barrier_sem, 2)

        my_half = my * t_half
        other_half = (1 - my) * t_half

        issue_gather(0)
        x_rdmas = []
        for k in range(N_CHUNKS):
            if k + 1 < N_CHUNKS:
                issue_gather(k + 1)
            wait_gather(k)
            sl = pl.ds(k * r, r)
            part_ref[sl, :] = part_ref[sl, :] * maskf_ref[sl, :]
            rdma = pltpu.make_async_remote_copy(
                src_ref=part_ref.at[sl],
                dst_ref=xrecv_ref.at[sl],
                send_sem=xsend_sems.at[k],
                recv_sem=xrecv_sems.at[k],
                device_id=x_nbr,
                device_id_type=pl.DeviceIdType.MESH,
            )
            rdma.start()
            x_rdmas.append(rdma)

        y_sends = []
        y_recvs = []
        for k in range(N_CHUNKS):
            x_rdmas[k].wait_recv()
            sl = pl.ds(k * r, r)
            out_sl = pl.ds(my_half + k * r, r)
            out_ref[out_sl, :] = part_ref[sl, :] + xrecv_ref[sl, :]
            send = pltpu.make_async_remote_copy(
                src_ref=out_ref.at[out_sl],
                dst_ref=out_ref.at[out_sl],
                send_sem=ysend_sems.at[k],
                recv_sem=yrecv_sems.at[k],
                device_id=y_nbr,
                device_id_type=pl.DeviceIdType.MESH,
            )
            send.start()
            y_sends.append(send)
            recv = pltpu.make_async_remote_copy(
                src_ref=out_ref.at[out_sl],
                dst_ref=out_ref.at[pl.ds(other_half + k * r, r)],
                send_sem=ysend_sems.at[k],
                recv_sem=yrecv_sems.at[k],
                device_id=y_nbr,
                device_id_type=pl.DeviceIdType.MESH,
            )
            y_recvs.append(recv)

        for k in range(N_CHUNKS):
            y_recvs[k].wait_recv()
        for k in range(N_CHUNKS):
            x_rdmas[k].wait_send()
            y_sends[k].wait_send()

    return pl.pallas_call(
        body,
        out_shape=jax.ShapeDtypeStruct((t, d), jnp.float32),
        in_specs=[
            pl.BlockSpec(memory_space=pltpu.SMEM),
            pl.BlockSpec(memory_space=pltpu.VMEM),
            pl.BlockSpec(memory_space=pl.ANY),
        ],
        out_specs=pl.BlockSpec(memory_space=pltpu.VMEM),
        scratch_shapes=[
            pltpu.VMEM((t_half, d), jnp.float32),
            pltpu.VMEM((t_half, d), jnp.float32),
            pltpu.SemaphoreType.DMA((N_CHUNKS,)),
            pltpu.SemaphoreType.DMA((N_CHUNKS,)),
            pltpu.SemaphoreType.DMA((N_CHUNKS,)),
            pltpu.SemaphoreType.DMA((N_CHUNKS,)),
            pltpu.SemaphoreType.DMA((N_CHUNKS,)),
        ],
        compiler_params=pltpu.CompilerParams(collective_id=0),
    )(clamped, maskf, E)


# device time: 79771 ns/iter; 1.7026x vs baseline; 1.0498x over previous
import jax
import jax.numpy as jnp
from jax import lax
from jax.experimental import pallas as pl
from jax.experimental.pallas import tpu as pltpu

N_CHUNKS = 8


def kernel(ids, E):
    v_loc, d = E.shape
    t = ids.shape[0]
    t_half = t // 2
    r = t_half // N_CHUNKS

    my_x = lax.axis_index("x")
    my_y = lax.axis_index("y")

    ids_half = lax.dynamic_slice(ids, (my_y * t_half,), (t_half,))
    local = ids_half - my_x * v_loc
    clamped = jnp.clip(local, 0, v_loc - 1).astype(jnp.int32)
    mask = (local >= 0) & (local < v_loc)
    maskf = mask.astype(jnp.float32)[:, None]

    miss = jnp.logical_not(mask).reshape(N_CHUNKS, r)
    order = jnp.argsort(miss, axis=1, stable=True)
    hit_dst = (order + (jnp.arange(N_CHUNKS) * r)[:, None]).reshape(-1)
    hit_dst = hit_dst.astype(jnp.int32)
    hit_src = jnp.take_along_axis(
        clamped.reshape(N_CHUNKS, r), order, axis=1
    ).reshape(-1).astype(jnp.int32)
    counts = mask.reshape(N_CHUNKS, r).sum(axis=1).astype(jnp.int32)

    def body(src_ref, dst_ref, cnt_ref, maskf_ref, e_ref, out_ref,
             part_ref, xrecv_ref,
             gather_sems, xsend_sems, xrecv_sems, ysend_sems, yrecv_sems):
        mx = lax.axis_index("x")
        my = lax.axis_index("y")
        x_nbr = (1 - mx, my)
        y_nbr = (mx, 1 - my)

        def row_copy(row_idx, dst_row, sem):
            return pltpu.make_async_copy(
                e_ref.at[pl.ds(row_idx, 1)],
                part_ref.at[pl.ds(dst_row, 1)],
                sem,
            )

        def issue_gather(k):
            def fi(i, carry):
                j = k * r + i
                row_copy(src_ref[j], dst_ref[j], gather_sems.at[k]).start()
                return carry
            lax.fori_loop(0, cnt_ref[k], fi, 0)

        def wait_gather(k):
            def fi(i, carry):
                row_copy(0, 0, gather_sems.at[k]).wait()
                return carry
            lax.fori_loop(0, cnt_ref[k], fi, 0)

        barrier_sem = pltpu.get_barrier_semaphore()
        for nbr in (x_nbr, y_nbr):
            pl.semaphore_signal(
                barrier_sem, inc=1,
                device_id=nbr, device_id_type=pl.DeviceIdType.MESH,
            )
        pl.semaphore_wait(barrier_sem, 2)

        my_half = my * t_half
        other_half = (1 - my) * t_half

        issue_gather(0)
        x_rdmas = []
        for k in range(N_CHUNKS):
            if k + 1 < N_CHUNKS:
                issue_gather(k + 1)
            wait_gather(k)
            sl = pl.ds(k * r, r)
            part_ref[sl, :] = jnp.where(
                maskf_ref[sl, :] > 0, part_ref[sl, :], jnp.float32(0)
            )
            rdma = pltpu.make_async_remote_copy(
                src_ref=part_ref.at[sl],
                dst_ref=xrecv_ref.at[sl],
                send_sem=xsend_sems.at[k],
                recv_sem=xrecv_sems.at[k],
                device_id=x_nbr,
                device_id_type=pl.DeviceIdType.MESH,
            )
            rdma.start()
            x_rdmas.append(rdma)

        y_sends = []
        y_recvs = []
        for k in range(N_CHUNKS):
            x_rdmas[k].wait_recv()
            sl = pl.ds(k * r, r)
            out_sl = pl.ds(my_half + k * r, r)
            out_ref[out_sl, :] = part_ref[sl, :] + xrecv_ref[sl, :]
            send = pltpu.make_async_remote_copy(
                src_ref=out_ref.at[out_sl],
                dst_ref=out_ref.at[out_sl],
                send_sem=ysend_sems.at[k],
                recv_sem=yrecv_sems.at[k],
                device_id=y_nbr,
                device_id_type=pl.DeviceIdType.MESH,
            )
            send.start()
            y_sends.append(send)
            recv = pltpu.make_async_remote_copy(
                src_ref=out_ref.at[out_sl],
                dst_ref=out_ref.at[pl.ds(other_half + k * r, r)],
                send_sem=ysend_sems.at[k],
                recv_sem=yrecv_sems.at[k],
                device_id=y_nbr,
                device_id_type=pl.DeviceIdType.MESH,
            )
            y_recvs.append(recv)

        for k in range(N_CHUNKS):
            y_recvs[k].wait_recv()
        for k in range(N_CHUNKS):
            x_rdmas[k].wait_send()
            y_sends[k].wait_send()

    return pl.pallas_call(
        body,
        out_shape=jax.ShapeDtypeStruct((t, d), jnp.float32),
        in_specs=[
            pl.BlockSpec(memory_space=pltpu.SMEM),
            pl.BlockSpec(memory_space=pltpu.SMEM),
            pl.BlockSpec(memory_space=pltpu.SMEM),
            pl.BlockSpec(memory_space=pltpu.VMEM),
            pl.BlockSpec(memory_space=pl.ANY),
        ],
        out_specs=pl.BlockSpec(memory_space=pltpu.VMEM),
        scratch_shapes=[
            pltpu.VMEM((t_half, d), jnp.float32),
            pltpu.VMEM((t_half, d), jnp.float32),
            pltpu.SemaphoreType.DMA((N_CHUNKS,)),
            pltpu.SemaphoreType.DMA((N_CHUNKS,)),
            pltpu.SemaphoreType.DMA((N_CHUNKS,)),
            pltpu.SemaphoreType.DMA((N_CHUNKS,)),
            pltpu.SemaphoreType.DMA((N_CHUNKS,)),
        ],
        compiler_params=pltpu.CompilerParams(collective_id=0),
    )(hit_src, hit_dst, counts, maskf, E)


# device time: 67945 ns/iter; 1.9989x vs baseline; 1.1741x over previous
import jax
import jax.numpy as jnp
from jax import lax
from jax.experimental import pallas as pl
from jax.experimental.pallas import tpu as pltpu

N_CHUNKS = 8
LAG = 2


def kernel(ids, E):
    v_loc, d = E.shape
    t = ids.shape[0]
    t_half = t // 2
    r = t_half // N_CHUNKS

    my_x = lax.axis_index("x")
    my_y = lax.axis_index("y")

    ids_half = lax.dynamic_slice(ids, (my_y * t_half,), (t_half,))
    local = (ids_half - my_x * v_loc).astype(jnp.int32)
    mask = (local >= 0) & (local < v_loc)
    maskf = mask.astype(jnp.float32)[:, None]
    counts = mask.reshape(N_CHUNKS, r).sum(axis=1).astype(jnp.int32)

    def body(local_ref, cnt_ref, maskf_ref, e_ref, out_ref,
             part_ref, xrecv_ref,
             gather_sems, xsend_sems, xrecv_sems, ysend_sems, yrecv_sems):
        mx = lax.axis_index("x")
        my = lax.axis_index("y")
        x_nbr = (1 - mx, my)
        y_nbr = (mx, 1 - my)

        def row_copy(row_idx, dst_row, sem):
            return pltpu.make_async_copy(
                e_ref.at[pl.ds(row_idx, 1)],
                part_ref.at[pl.ds(dst_row, 1)],
                sem,
            )

        def issue_gather(k):
            def fi(i, carry):
                j = k * r + i
                v = local_ref[j]
                ok = (v >= 0) & (v < v_loc)

                @pl.when(ok)
                def _():
                    row_copy(v, j, gather_sems.at[k]).start()

                return carry
            lax.fori_loop(0, r, fi, 0)

        def wait_gather(k):
            def fi(i, carry):
                row_copy(0, 0, gather_sems.at[k]).wait()
                return carry
            lax.fori_loop(0, cnt_ref[k], fi, 0)

        barrier_sem = pltpu.get_barrier_semaphore()
        for nbr in (x_nbr, y_nbr):
            pl.semaphore_signal(
                barrier_sem, inc=1,
                device_id=nbr, device_id_type=pl.DeviceIdType.MESH,
            )
        pl.semaphore_wait(barrier_sem, 2)

        my_half = my * t_half
        other_half = (1 - my) * t_half

        x_rdmas = [None] * N_CHUNKS
        y_sends = [None] * N_CHUNKS
        y_recvs = [None] * N_CHUNKS

        def phase1(k):
            if k + 1 < N_CHUNKS:
                issue_gather(k + 1)
            wait_gather(k)
            sl = pl.ds(k * r, r)
            rdma = pltpu.make_async_remote_copy(
                src_ref=part_ref.at[sl],
                dst_ref=xrecv_ref.at[sl],
                send_sem=xsend_sems.at[k],
                recv_sem=xrecv_sems.at[k],
                device_id=x_nbr,
                device_id_type=pl.DeviceIdType.MESH,
            )
            rdma.start()
            x_rdmas[k] = rdma

        def phase2(k):
            x_rdmas[k].wait_recv()
            sl = pl.ds(k * r, r)
            out_sl = pl.ds(my_half + k * r, r)
            out_ref[out_sl, :] = jnp.where(
                maskf_ref[sl, :] > 0, part_ref[sl, :], xrecv_ref[sl, :]
            )
            send = pltpu.make_async_remote_copy(
                src_ref=out_ref.at[out_sl],
                dst_ref=out_ref.at[out_sl],
                send_sem=ysend_sems.at[k],
                recv_sem=yrecv_sems.at[k],
                device_id=y_nbr,
                device_id_type=pl.DeviceIdType.MESH,
            )
            send.start()
            y_sends[k] = send
            y_recvs[k] = pltpu.make_async_remote_copy(
                src_ref=out_ref.at[out_sl],
                dst_ref=out_ref.at[pl.ds(other_half + k * r, r)],
                send_sem=ysend_sems.at[k],
                recv_sem=yrecv_sems.at[k],
                device_id=y_nbr,
                device_id_type=pl.DeviceIdType.MESH,
            )

        issue_gather(0)
        for k in range(N_CHUNKS):
            phase1(k)
            if k >= LAG:
                phase2(k - LAG)
        for k in range(N_CHUNKS - LAG, N_CHUNKS):
            phase2(k)

        for k in range(N_CHUNKS):
            y_recvs[k].wait_recv()
        for k in range(N_CHUNKS):
            x_rdmas[k].wait_send()
            y_sends[k].wait_send()

    return pl.pallas_call(
        body,
        out_shape=jax.ShapeDtypeStruct((t, d), jnp.float32),
        in_specs=[
            pl.BlockSpec(memory_space=pltpu.SMEM),
            pl.BlockSpec(memory_space=pltpu.SMEM),
            pl.BlockSpec(memory_space=pltpu.VMEM),
            pl.BlockSpec(memory_space=pl.ANY),
        ],
        out_specs=pl.BlockSpec(memory_space=pltpu.VMEM),
        scratch_shapes=[
            pltpu.VMEM((t_half, d), jnp.float32),
            pltpu.VMEM((t_half, d), jnp.float32),
            pltpu.SemaphoreType.DMA((N_CHUNKS,)),
            pltpu.SemaphoreType.DMA((N_CHUNKS,)),
            pltpu.SemaphoreType.DMA((N_CHUNKS,)),
            pltpu.SemaphoreType.DMA((N_CHUNKS,)),
            pltpu.SemaphoreType.DMA((N_CHUNKS,)),
        ],
        compiler_params=pltpu.CompilerParams(collective_id=0),
    )(local, counts, maskf, E)


# device time: 63993 ns/iter; 2.1223x vs baseline; 1.0618x over previous
import jax
import jax.numpy as jnp
from jax import lax
from jax.experimental import pallas as pl
from jax.experimental.pallas import tpu as pltpu

N_CHUNKS = 16
LAG = 3
PREFETCH = 2


def kernel(ids, E):
    v_loc, d = E.shape
    t = ids.shape[0]
    t_half = t // 2
    r = t_half // N_CHUNKS

    my_x = lax.axis_index("x")
    my_y = lax.axis_index("y")

    ids_half = lax.dynamic_slice(ids, (my_y * t_half,), (t_half,))
    local = (ids_half - my_x * v_loc).astype(jnp.int32)
    mask = (local >= 0) & (local < v_loc)
    maskf = mask.astype(jnp.float32)[:, None]
    counts = mask.reshape(N_CHUNKS, r).sum(axis=1).astype(jnp.int32)

    def body(local_ref, cnt_ref, maskf_ref, e_ref, out_ref,
             part_ref, xrecv_ref,
             gather_sems, xsend_sems, xrecv_sems, ysend_sems, yrecv_sems):
        mx = lax.axis_index("x")
        my = lax.axis_index("y")
        x_nbr = (1 - mx, my)
        y_nbr = (mx, 1 - my)

        def row_copy(row_idx, dst_row, sem):
            return pltpu.make_async_copy(
                e_ref.at[pl.ds(row_idx, 1)],
                part_ref.at[pl.ds(dst_row, 1)],
                sem,
            )

        def issue_gather(k):
            def fi(i, carry):
                j = k * r + i
                v = local_ref[j]
                ok = (v >= 0) & (v < v_loc)

                @pl.when(ok)
                def _():
                    row_copy(v, j, gather_sems.at[k]).start()

                return carry
            lax.fori_loop(0, r, fi, 0)

        def wait_gather(k):
            def fi(i, carry):
                row_copy(0, 0, gather_sems.at[k]).wait()
                return carry
            lax.fori_loop(0, cnt_ref[k], fi, 0)

        for k in range(PREFETCH):
            issue_gather(k)

        barrier_sem = pltpu.get_barrier_semaphore()
        for nbr in (x_nbr, y_nbr):
            pl.semaphore_signal(
                barrier_sem, inc=1,
                device_id=nbr, device_id_type=pl.DeviceIdType.MESH,
            )
        pl.semaphore_wait(barrier_sem, 2)

        my_half = my * t_half
        other_half = (1 - my) * t_half

        x_rdmas = [None] * N_CHUNKS
        y_sends = [None] * N_CHUNKS
        y_recvs = [None] * N_CHUNKS

        def phase1(k):
            if k + PREFETCH < N_CHUNKS:
                issue_gather(k + PREFETCH)
            wait_gather(k)
            sl = pl.ds(k * r, r)
            rdma = pltpu.make_async_remote_copy(
                src_ref=part_ref.at[sl],
                dst_ref=xrecv_ref.at[sl],
                send_sem=xsend_sems.at[k],
                recv_sem=xrecv_sems.at[k],
                device_id=x_nbr,
                device_id_type=pl.DeviceIdType.MESH,
            )
            rdma.start()
            x_rdmas[k] = rdma

        def phase2(k):
            x_rdmas[k].wait_recv()
            sl = pl.ds(k * r, r)
            out_sl = pl.ds(my_half + k * r, r)
            out_ref[out_sl, :] = jnp.where(
                maskf_ref[sl, :] > 0, part_ref[sl, :], xrecv_ref[sl, :]
            )
            send = pltpu.make_async_remote_copy(
                src_ref=out_ref.at[out_sl],
                dst_ref=out_ref.at[out_sl],
                send_sem=ysend_sems.at[k],
                recv_sem=yrecv_sems.at[k],
                device_id=y_nbr,
                device_id_type=pl.DeviceIdType.MESH,
            )
            send.start()
            y_sends[k] = send
            y_recvs[k] = pltpu.make_async_remote_copy(
                src_ref=out_ref.at[out_sl],
                dst_ref=out_ref.at[pl.ds(other_half + k * r, r)],
                send_sem=ysend_sems.at[k],
                recv_sem=yrecv_sems.at[k],
                device_id=y_nbr,
                device_id_type=pl.DeviceIdType.MESH,
            )

        for k in range(N_CHUNKS):
            phase1(k)
            if k >= LAG:
                phase2(k - LAG)
        for k in range(N_CHUNKS - LAG, N_CHUNKS):
            phase2(k)

        for k in range(N_CHUNKS):
            y_recvs[k].wait_recv()
        for k in range(N_CHUNKS):
            x_rdmas[k].wait_send()
            y_sends[k].wait_send()

    return pl.pallas_call(
        body,
        out_shape=jax.ShapeDtypeStruct((t, d), jnp.float32),
        in_specs=[
            pl.BlockSpec(memory_space=pltpu.SMEM),
            pl.BlockSpec(memory_space=pltpu.SMEM),
            pl.BlockSpec(memory_space=pltpu.VMEM),
            pl.BlockSpec(memory_space=pl.ANY),
        ],
        out_specs=pl.BlockSpec(memory_space=pltpu.VMEM),
        scratch_shapes=[
            pltpu.VMEM((t_half, d), jnp.float32),
            pltpu.VMEM((t_half, d), jnp.float32),
            pltpu.SemaphoreType.DMA((N_CHUNKS,)),
            pltpu.SemaphoreType.DMA((N_CHUNKS,)),
            pltpu.SemaphoreType.DMA((N_CHUNKS,)),
            pltpu.SemaphoreType.DMA((N_CHUNKS,)),
            pltpu.SemaphoreType.DMA((N_CHUNKS,)),
        ],
        compiler_params=pltpu.CompilerParams(collective_id=0),
    )(local, counts, maskf, E)


# device time: 62064 ns/iter; 2.1883x vs baseline; 1.0311x over previous
import jax
import jax.numpy as jnp
from jax import lax
from jax.experimental import pallas as pl
from jax.experimental.pallas import tpu as pltpu

N_CHUNKS = 16
LAG = 3
PREFETCH = 2


def kernel(ids, E):
    v_loc, d = E.shape
    t = ids.shape[0]
    t_half = t // 2
    r = t_half // N_CHUNKS

    my_x = lax.axis_index("x")
    my_y = lax.axis_index("y")

    ids_half = lax.dynamic_slice(ids, (my_y * t_half,), (t_half,))
    local = (ids_half - my_x * v_loc).astype(jnp.int32)
    mask = (local >= 0) & (local < v_loc)
    maskf = mask.astype(jnp.float32)[:, None]

    def body(local_ref, maskf_ref, e_ref, out_ref,
             part_ref, xrecv_ref,
             gather_sems, xsend_sems, xrecv_sems, ysend_sems, yrecv_sems):
        mx = lax.axis_index("x")
        my = lax.axis_index("y")
        x_nbr = (1 - mx, my)
        y_nbr = (mx, 1 - my)

        def row_copy(row_idx, dst_row, sem):
            return pltpu.make_async_copy(
                e_ref.at[pl.ds(row_idx, 1)],
                part_ref.at[pl.ds(dst_row, 1)],
                sem,
            )

        def issue_gather(k):
            def fi(i, carry):
                j = k * r + i
                v = local_ref[j]
                c = jnp.maximum(jnp.minimum(v, v_loc - 1), 0)
                row_copy(c, j, gather_sems.at[k]).start()
                return carry
            lax.fori_loop(0, r, fi, 0, unroll=8)

        def wait_gather(k):
            pltpu.make_async_copy(
                e_ref.at[pl.ds(0, r)],
                part_ref.at[pl.ds(k * r, r)],
                gather_sems.at[k],
            ).wait()

        for k in range(PREFETCH):
            issue_gather(k)

        barrier_sem = pltpu.get_barrier_semaphore()
        for nbr in (x_nbr, y_nbr):
            pl.semaphore_signal(
                barrier_sem, inc=1,
                device_id=nbr, device_id_type=pl.DeviceIdType.MESH,
            )
        pl.semaphore_wait(barrier_sem, 2)

        my_half = my * t_half
        other_half = (1 - my) * t_half

        x_rdmas = [None] * N_CHUNKS
        y_sends = [None] * N_CHUNKS
        y_recvs = [None] * N_CHUNKS

        def phase1(k):
            if k + PREFETCH < N_CHUNKS:
                issue_gather(k + PREFETCH)
            wait_gather(k)
            sl = pl.ds(k * r, r)
            rdma = pltpu.make_async_remote_copy(
                src_ref=part_ref.at[sl],
                dst_ref=xrecv_ref.at[sl],
                send_sem=xsend_sems.at[k],
                recv_sem=xrecv_sems.at[k],
                device_id=x_nbr,
                device_id_type=pl.DeviceIdType.MESH,
            )
            rdma.start()
            x_rdmas[k] = rdma

        def phase2(k):
            x_rdmas[k].wait_recv()
            sl = pl.ds(k * r, r)
            out_sl = pl.ds(my_half + k * r, r)
            out_ref[out_sl, :] = jnp.where(
                maskf_ref[sl, :] > 0, part_ref[sl, :], xrecv_ref[sl, :]
            )
            send = pltpu.make_async_remote_copy(
                src_ref=out_ref.at[out_sl],
                dst_ref=out_ref.at[out_sl],
                send_sem=ysend_sems.at[k],
                recv_sem=yrecv_sems.at[k],
                device_id=y_nbr,
                device_id_type=pl.DeviceIdType.MESH,
            )
            send.start()
            y_sends[k] = send
            y_recvs[k] = pltpu.make_async_remote_copy(
                src_ref=out_ref.at[out_sl],
                dst_ref=out_ref.at[pl.ds(other_half + k * r, r)],
                send_sem=ysend_sems.at[k],
                recv_sem=yrecv_sems.at[k],
                device_id=y_nbr,
                device_id_type=pl.DeviceIdType.MESH,
            )

        for k in range(N_CHUNKS):
            phase1(k)
            if k >= LAG:
                phase2(k - LAG)
        for k in range(N_CHUNKS - LAG, N_CHUNKS):
            phase2(k)

        for k in range(N_CHUNKS):
            y_recvs[k].wait_recv()
        for k in range(N_CHUNKS):
            x_rdmas[k].wait_send()
            y_sends[k].wait_send()

    return pl.pallas_call(
        body,
        out_shape=jax.ShapeDtypeStruct((t, d), jnp.float32),
        in_specs=[
            pl.BlockSpec(memory_space=pltpu.SMEM),
            pl.BlockSpec(memory_space=pltpu.VMEM),
            pl.BlockSpec(memory_space=pl.ANY),
        ],
        out_specs=pl.BlockSpec(memory_space=pltpu.VMEM),
        scratch_shapes=[
            pltpu.VMEM((t_half, d), jnp.float32),
            pltpu.VMEM((t_half, d), jnp.float32),
            pltpu.SemaphoreType.DMA((N_CHUNKS,)),
            pltpu.SemaphoreType.DMA((N_CHUNKS,)),
            pltpu.SemaphoreType.DMA((N_CHUNKS,)),
            pltpu.SemaphoreType.DMA((N_CHUNKS,)),
            pltpu.SemaphoreType.DMA((N_CHUNKS,)),
        ],
        compiler_params=pltpu.CompilerParams(collective_id=0),
    )(local, maskf, E)
